# baseline (speedup 1.0000x reference)
import functools

import jax
import jax.numpy as jnp
from jax.experimental import pallas as pl


_TAPS = (
    ((0, 0), 0, 0, 0),
    ((0, 1), 1, 0, 0),
    ((0, 2), 0, 0, 1),
    ((1, 0), 2, 0, 0),
    ((1, 1), 3, 0, 0),
    ((1, 2), 2, 0, 1),
    ((2, 0), 0, 1, 0),
    ((2, 1), 1, 1, 0),
    ((2, 2), 0, 1, 1),
)


def _phase_split(x):
    e = x[:, 0::2]
    o = x[:, 1::2]
    ee = e[:, :, 0::2]
    eo = e[:, :, 1::2]
    oe = o[:, :, 0::2]
    oo = o[:, :, 1::2]
    ee = jnp.pad(ee, ((0, 0), (0, 1), (0, 1), (0, 0)))
    eo = jnp.pad(eo, ((0, 0), (0, 1), (0, 0), (0, 0)))
    oe = jnp.pad(oe, ((0, 0), (0, 0), (0, 1), (0, 0)))
    return ee, eo, oe, oo


def _tap_slice(phase_refs, pid, r0, nrows, ro, co, wo, cin):
    ph = phase_refs[pid]
    x = ph[0, r0 + ro:r0 + ro + nrows, co:co + wo, :]
    return x.reshape(nrows * wo, cin)


def _conv_body(ee_ref, eo_ref, oe_ref, oo_ref, w_ref, b_ref, out_ref,
               *, tr, ho, wo, cin, cout):
    for r in range(ho // tr):
        acc = jnp.zeros((tr * wo, cout), dtype=jnp.float32)
        for t, (_, pid, ro, co) in enumerate(_TAPS):
            x = _tap_slice((ee_ref, eo_ref, oe_ref, oo_ref), pid,
                           r * tr, tr, ro, co, wo, cin)
            acc = acc + jnp.dot(x, w_ref[t],
                                preferred_element_type=jnp.float32)
        y = jnp.maximum(acc + b_ref[...], 0.0)
        out_ref[0, r * tr:(r + 1) * tr, :, :] = y.reshape(tr, wo, cout)


def _conv_layer(phases, w_taps, b, *, tr, ho, wo, cin, cout, batch):
    shapes = [(1, ho + 1, wo + 1, cin), (1, ho + 1, wo, cin),
              (1, ho, wo + 1, cin), (1, ho, wo, cin)]
    in_specs = [pl.BlockSpec(s, lambda i: (i, 0, 0, 0)) for s in shapes]
    in_specs.append(pl.BlockSpec((9, cin, cout), lambda i: (0, 0, 0)))
    in_specs.append(pl.BlockSpec((1, cout), lambda i: (0, 0)))
    body = functools.partial(_conv_body, tr=tr, ho=ho, wo=wo, cin=cin,
                             cout=cout)
    return pl.pallas_call(
        body,
        grid=(batch,),
        in_specs=in_specs,
        out_specs=pl.BlockSpec((1, ho, wo, cout), lambda i: (i, 0, 0, 0)),
        out_shape=jax.ShapeDtypeStruct((batch, ho, wo, cout), jnp.float32),
    )(*phases, w_taps, b.reshape(1, cout))


def _conv3_body(ee_ref, eo_ref, oe_ref, oo_ref, w_ref, b_ref, out_ref,
                *, bt, ho, wo, cin, cout, nsplit):
    cn = cout // nsplit
    m = ho * wo
    for i in range(bt):
        for n in range(nsplit):
            acc = jnp.zeros((m, cn), dtype=jnp.float32)
            for t, (_, pid, ro, co) in enumerate(_TAPS):
                ph = (ee_ref, eo_ref, oe_ref, oo_ref)[pid]
                x = ph[i, ro:ro + ho, co:co + wo, :].reshape(m, cin)
                acc = acc + jnp.dot(x, w_ref[t, :, n * cn:(n + 1) * cn],
                                    preferred_element_type=jnp.float32)
            y = jnp.maximum(acc + b_ref[0, n * cn:(n + 1) * cn], 0.0)
            out_ref[i:i + 1, n * cn:(n + 1) * cn] = (
                jnp.sum(y, axis=0, keepdims=True) * (1.0 / m))


def _conv0_body(p_ref, w_ref, b_ref, out_ref, *, m, kdim, cout, tiles):
    tm = m // tiles
    for r in range(tiles):
        x = p_ref[0, r * tm:(r + 1) * tm, :]
        y = jnp.dot(x, w_ref[...], preferred_element_type=jnp.float32)
        y = jnp.maximum(y + b_ref[...], 0.0)
        out_ref[0, r * tm:(r + 1) * tm, :] = y


def _vq_body(f_ref, cb_ref, out_ref, *, batch, cdim, k, kc):
    flat = f_ref[...]
    nchunk = k // kc
    dcols = []
    for c in range(nchunk):
        cbc = cb_ref[c * kc:(c + 1) * kc, :]
        csq = jnp.sum(cbc * cbc, axis=1)
        prod = jax.lax.dot_general(
            flat, cbc, (((1,), (1,)), ((), ())),
            preferred_element_type=jnp.float32)
        dcols.append(csq[None, :] - 2.0 * prod)
    dist = jnp.concatenate(dcols, axis=1)
    m = jnp.min(dist, axis=1, keepdims=True)
    iota = jax.lax.broadcasted_iota(jnp.int32, (batch, k), 1)
    idx = jnp.min(jnp.where(dist == m, iota, k), axis=1, keepdims=True)
    onehot = (iota == idx).astype(jnp.float32)
    acc = jnp.zeros((batch, cdim), dtype=jnp.float32)
    for c in range(nchunk):
        cbc = cb_ref[c * kc:(c + 1) * kc, :]
        acc = acc + jnp.dot(onehot[:, c * kc:(c + 1) * kc], cbc,
                            preferred_element_type=jnp.float32)
    out_ref[...] = acc


def kernel(inputs, W0, b0, W1, b1, W2, b2, W3, b3, codebook):
    batch = inputs.shape[0]

    def w_taps(w):
        return jnp.transpose(w, (2, 3, 1, 0)).reshape(9, w.shape[1], w.shape[0])

    x = jnp.transpose(inputs, (0, 2, 3, 1))

    ph0 = _phase_split(x)
    cols = []
    for _, pid, ro, co in _TAPS:
        p = ph0[pid]
        cols.append(p[:, ro:ro + 112, co:co + 112, :])
    patches = jnp.concatenate(cols, axis=-1).reshape(batch, 112 * 112, 27)
    w0 = jnp.transpose(W0, (2, 3, 1, 0)).reshape(27, 96)
    f0 = pl.pallas_call(
        functools.partial(_conv0_body, m=112 * 112, kdim=27, cout=96,
                          tiles=16),
        grid=(batch,),
        in_specs=[pl.BlockSpec((1, 112 * 112, 27), lambda i: (i, 0, 0)),
                  pl.BlockSpec((27, 96), lambda i: (0, 0)),
                  pl.BlockSpec((1, 96), lambda i: (0, 0))],
        out_specs=pl.BlockSpec((1, 112 * 112, 96), lambda i: (i, 0, 0)),
        out_shape=jax.ShapeDtypeStruct((batch, 112 * 112, 96), jnp.float32),
    )(patches, w0, b0.reshape(1, 96))
    f0 = f0.reshape(batch, 112, 112, 96)

    f1 = _conv_layer(_phase_split(f0), w_taps(W1), b1, tr=8, ho=56, wo=56,
                     cin=96, cout=192, batch=batch)
    f2 = _conv_layer(_phase_split(f1), w_taps(W2), b2, tr=7, ho=28, wo=28,
                     cin=192, cout=384, batch=batch)

    ph3 = _phase_split(f2)
    bt = 8
    shapes = [(bt, 15, 15, 384), (bt, 15, 14, 384),
              (bt, 14, 15, 384), (bt, 14, 14, 384)]
    in_specs = [pl.BlockSpec(s, lambda i: (i, 0, 0, 0)) for s in shapes]
    in_specs.append(pl.BlockSpec((9, 384, 768), lambda i: (0, 0, 0)))
    in_specs.append(pl.BlockSpec((1, 768), lambda i: (0, 0)))
    flat = pl.pallas_call(
        functools.partial(_conv3_body, bt=bt, ho=14, wo=14, cin=384,
                          cout=768, nsplit=2),
        grid=(batch // bt,),
        in_specs=in_specs,
        out_specs=pl.BlockSpec((bt, 768), lambda i: (i, 0)),
        out_shape=jax.ShapeDtypeStruct((batch, 768), jnp.float32),
    )(*ph3, w_taps(W3), b3.reshape(1, 768))

    k, cdim = codebook.shape
    quant = pl.pallas_call(
        functools.partial(_vq_body, batch=batch, cdim=cdim, k=k, kc=128),
        in_specs=[pl.BlockSpec((batch, cdim), lambda: (0, 0)),
                  pl.BlockSpec((k, cdim), lambda: (0, 0))],
        out_specs=pl.BlockSpec((batch, cdim), lambda: (0, 0)),
        out_shape=jax.ShapeDtypeStruct((batch, cdim), jnp.float32),
    )(flat, codebook)
    return quant.reshape(batch, cdim, 1, 1)
